# baseline (speedup 1.0000x reference)
import jax
import jax.numpy as jnp
from jax.experimental import pallas as pl
from jax.experimental.pallas import tpu as pltpu


_CHUNK_ROWS = 1024


def _copy_body(src_hbm, dst_hbm, buf, in_sems, out_sems):
    rows = src_hbm.shape[0]
    n = rows // _CHUNK_ROWS

    def in_copy(i, slot):
        return pltpu.make_async_copy(
            src_hbm.at[pl.ds(i * _CHUNK_ROWS, _CHUNK_ROWS)], buf.at[slot],
            in_sems.at[i])

    def out_copy(i, slot):
        return pltpu.make_async_copy(
            buf.at[slot], dst_hbm.at[0, pl.ds(i * _CHUNK_ROWS, _CHUNK_ROWS)],
            out_sems.at[i])

    in_copy(0, 0).start()
    for i in range(n):
        slot = i % 2
        in_copy(i, slot).wait()
        out_copy(i, slot).start()
        if i + 1 < n:
            nslot = (i + 1) % 2
            if i - 1 >= 0:
                out_copy(i - 1, nslot).wait()
            in_copy(i + 1, nslot).start()
    out_copy(n - 2, (n - 2) % 2).wait()
    out_copy(n - 1, (n - 1) % 2).wait()


def kernel(x, pe_weight):
    seq_len = x.shape[1]
    hidden = pe_weight.shape[1]
    n = seq_len // _CHUNK_ROWS
    return pl.pallas_call(
        _copy_body,
        out_shape=jax.ShapeDtypeStruct((1, seq_len, hidden), pe_weight.dtype),
        in_specs=[pl.BlockSpec(memory_space=pl.ANY)],
        out_specs=pl.BlockSpec(memory_space=pl.ANY),
        scratch_shapes=[
            pltpu.VMEM((2, _CHUNK_ROWS, hidden), pe_weight.dtype),
            pltpu.SemaphoreType.DMA((n,)),
            pltpu.SemaphoreType.DMA((n,)),
        ],
    )(pe_weight)


# speedup vs baseline: 1.4687x; 1.4687x over previous
import jax
import jax.numpy as jnp
from jax.experimental import pallas as pl
from jax.experimental.pallas import tpu as pltpu


_CHUNK_ROWS = 1024


def _copy_body(src_hbm, dst_hbm, buf, in_sems, out_sems):
    rows = src_hbm.shape[0]
    n = rows // _CHUNK_ROWS

    def in_copy(i):
        return pltpu.make_async_copy(
            src_hbm.at[pl.ds(i * _CHUNK_ROWS, _CHUNK_ROWS)], buf.at[i],
            in_sems.at[i])

    def out_copy(i):
        return pltpu.make_async_copy(
            buf.at[i], dst_hbm.at[0, pl.ds(i * _CHUNK_ROWS, _CHUNK_ROWS)],
            out_sems.at[i])

    for i in range(n):
        in_copy(i).start()
    for i in range(n):
        in_copy(i).wait()
        out_copy(i).start()
    for i in range(n):
        out_copy(i).wait()


def kernel(x, pe_weight):
    seq_len = x.shape[1]
    hidden = pe_weight.shape[1]
    n = seq_len // _CHUNK_ROWS
    return pl.pallas_call(
        _copy_body,
        out_shape=jax.ShapeDtypeStruct((1, seq_len, hidden), pe_weight.dtype),
        in_specs=[pl.BlockSpec(memory_space=pl.ANY)],
        out_specs=pl.BlockSpec(memory_space=pl.ANY),
        scratch_shapes=[
            pltpu.VMEM((n, _CHUNK_ROWS, hidden), pe_weight.dtype),
            pltpu.SemaphoreType.DMA((n,)),
            pltpu.SemaphoreType.DMA((n,)),
        ],
    )(pe_weight)
